# Initial kernel scaffold; baseline (speedup 1.0000x reference)
import jax
import jax.numpy as jnp
from jax.experimental import pallas as pl


def kernel(indices, W):
    raise NotImplementedError("write your pallas kernel here")


# speedup vs baseline: 2.1374x; 2.1374x over previous
import functools

import jax
import jax.numpy as jnp
from jax import lax
from jax.experimental import pallas as pl
from jax.experimental.pallas import tpu as pltpu
from jax.experimental.pallas import tpu_sc as plsc

B = 16384
L = 50
D = 64
BAGS_PER_CHUNK = 2
IDX_PER_CHUNK = BAGS_PER_CHUNK * L


def _make_kernel(n_workers):
    bags_per_w = B // n_workers
    chunks_per_w = bags_per_w // BAGS_PER_CHUNK
    mesh = plsc.VectorSubcoreMesh(core_axis_name="c", subcore_axis_name="s")

    @functools.partial(
        pl.kernel,
        mesh=mesh,
        out_type=jax.ShapeDtypeStruct((B, D), jnp.float32),
        compiler_params=pltpu.CompilerParams(use_tc_tiling_on_sc=False),
        scratch_types=[
            pltpu.VMEM((chunks_per_w, IDX_PER_CHUNK), jnp.int32),
            pltpu.VMEM((IDX_PER_CHUNK, D), jnp.float32),
            pltpu.VMEM((bags_per_w, D), jnp.float32),
            pltpu.SemaphoreType.DMA,
        ],
    )
    def embag(idx_hbm, w_hbm, out_hbm, idx_v, rows_v, out_v, sem):
        n_cores = lax.axis_size("c")
        wid = lax.axis_index("s") * n_cores + lax.axis_index("c")

        pltpu.sync_copy(
            idx_hbm.at[pl.ds(wid * chunks_per_w, chunks_per_w), :], idx_v
        )

        def chunk_body(c, _):
            pltpu.async_copy(w_hbm.at[idx_v.at[c]], rows_v, sem).wait()
            for b in range(BAGS_PER_CHUNK):
                base = b * L

                def row_body(r, acc):
                    return tuple(
                        acc[d] + rows_v[base + r, pl.ds(d * 16, 16)]
                        for d in range(D // 16)
                    )

                init = tuple(
                    rows_v[base, pl.ds(d * 16, 16)] for d in range(D // 16)
                )
                acc = lax.fori_loop(1, L, row_body, init)
                orow = c * BAGS_PER_CHUNK + b
                for d in range(D // 16):
                    out_v[orow, pl.ds(d * 16, 16)] = acc[d]
            return ()

        lax.fori_loop(0, chunks_per_w, chunk_body, ())

        pltpu.sync_copy(
            out_v, out_hbm.at[pl.ds(wid * bags_per_w, bags_per_w), :]
        )

    return embag


@jax.jit
def kernel(indices, W):
    info = plsc.get_sparse_core_info()
    n_workers = info.num_cores * info.num_subcores
    idx2 = jnp.reshape(indices.astype(jnp.int32), (B * L // IDX_PER_CHUNK, IDX_PER_CHUNK))
    return _make_kernel(n_workers)(idx2, W)


# speedup vs baseline: 2.5405x; 1.1886x over previous
import functools

import jax
import jax.numpy as jnp
from jax import lax
from jax.experimental import pallas as pl
from jax.experimental.pallas import tpu as pltpu
from jax.experimental.pallas import tpu_sc as plsc

B = 16384
L = 50
D = 64
BAGS_PER_CHUNK = 2
IDX_PER_CHUNK = BAGS_PER_CHUNK * L
RING = 2


def _make_kernel(n_workers):
    bags_per_w = B // n_workers
    n_chunks = bags_per_w // BAGS_PER_CHUNK
    mesh = plsc.VectorSubcoreMesh(core_axis_name="c", subcore_axis_name="s")

    @functools.partial(
        pl.kernel,
        mesh=mesh,
        out_type=jax.ShapeDtypeStruct((B, D), jnp.float32),
        compiler_params=pltpu.CompilerParams(use_tc_tiling_on_sc=False),
        scratch_types=[
            pltpu.VMEM((n_chunks, IDX_PER_CHUNK), jnp.int32),
            pltpu.VMEM((RING, IDX_PER_CHUNK, D), jnp.float32),
            pltpu.VMEM((bags_per_w, D), jnp.float32),
        ]
        + [pltpu.SemaphoreType.DMA] * RING,
    )
    def embag(idx_hbm, w_hbm, out_hbm, idx_v, rows_v, out_v, *sems):
        n_cores = lax.axis_size("c")
        wid = lax.axis_index("s") * n_cores + lax.axis_index("c")

        pltpu.sync_copy(idx_hbm.at[pl.ds(wid * n_chunks, n_chunks), :], idx_v)

        for b in range(RING):
            pltpu.async_copy(w_hbm.at[idx_v.at[b]], rows_v.at[b], sems[b])

        def group_body(p, _):
            for b in range(RING):
                c = p * RING + b
                pltpu.make_async_copy(
                    w_hbm.at[idx_v.at[c]], rows_v.at[b], sems[b]
                ).wait()
                for bag in range(BAGS_PER_CHUNK):
                    base = bag * L
                    acc = [
                        rows_v[b, base, pl.ds(d * 16, 16)]
                        for d in range(D // 16)
                    ]
                    for r in range(1, L):
                        for d in range(D // 16):
                            acc[d] = acc[d] + rows_v[b, base + r, pl.ds(d * 16, 16)]
                    orow = c * BAGS_PER_CHUNK + bag
                    for d in range(D // 16):
                        out_v[orow, pl.ds(d * 16, 16)] = acc[d]

                @pl.when(c + RING < n_chunks)
                def _():
                    pltpu.async_copy(
                        w_hbm.at[idx_v.at[c + RING]], rows_v.at[b], sems[b]
                    )

            return ()

        lax.fori_loop(0, n_chunks // RING, group_body, ())

        pltpu.sync_copy(
            out_v, out_hbm.at[pl.ds(wid * bags_per_w, bags_per_w), :]
        )

    return embag


@jax.jit
def kernel(indices, W):
    info = plsc.get_sparse_core_info()
    n_workers = info.num_cores * info.num_subcores
    idx2 = jnp.reshape(indices.astype(jnp.int32), (B * L // IDX_PER_CHUNK, IDX_PER_CHUNK))
    return _make_kernel(n_workers)(idx2, W)
